# baseline (speedup 1.0000x reference)
import functools

import jax
import jax.numpy as jnp
from jax import lax
from jax.experimental import pallas as pl
from jax.experimental.pallas import tpu as pltpu
from jax.experimental.pallas import tpu_sc as plsc


def _tr_kernel(bank_ref, outbank_ref):
    outbank_ref[...] = bank_ref[...].T


def _make_sc_copy(size, dim, batch):
    info = plsc.get_sparse_core_info()
    nw = info.num_cores * info.num_subcores
    rows = size // nw
    mesh = plsc.VectorSubcoreMesh(core_axis_name="c", subcore_axis_name="s")

    @functools.partial(
        pl.kernel,
        mesh=mesh,
        out_type=jax.ShapeDtypeStruct((size, dim), jnp.float32),
        scratch_types=[pltpu.VMEM((16,), jnp.int32)],
    )
    def sc_copy(u_hbm, out_hbm, bank_hbm, new_hbm, u_v):
        wid = lax.axis_index("s") * info.num_cores + lax.axis_index("c")
        base = wid * rows
        pltpu.sync_copy(u_hbm, u_v)
        upd = u_v[...][0]

        def from_output():
            pltpu.sync_copy(out_hbm.at[pl.ds(base, rows)],
                            new_hbm.at[pl.ds(base, rows)])

        def from_bank():
            pltpu.sync_copy(bank_hbm.at[pl.ds(base, rows)],
                            new_hbm.at[pl.ds(base, rows)])

        lax.cond((upd != 0) & (base < batch), from_output, from_bank)

    return sc_copy


def kernel(output, bank, update):
    size, dim = bank.shape
    batch = output.shape[0]
    u16 = jnp.full((16,), jnp.asarray(update, jnp.int32))
    new_bank = _make_sc_copy(size, dim, batch)(u16, output, bank)
    r = 2048
    out_bank = pl.pallas_call(
        _tr_kernel,
        grid=(size // r,),
        in_specs=[pl.BlockSpec((r, dim), lambda i: (i, 0))],
        out_specs=pl.BlockSpec((dim, r), lambda i: (0, i)),
        out_shape=jax.ShapeDtypeStruct((dim, size), bank.dtype),
    )(bank)
    return (output, out_bank, new_bank)


# speedup vs baseline: 14.9324x; 14.9324x over previous
import functools

import jax
import jax.numpy as jnp
from jax import lax
from jax.experimental import pallas as pl
from jax.experimental.pallas import tpu as pltpu
from jax.experimental.pallas import tpu_sc as plsc


def _tr_kernel(bank_ref, outbank_ref):
    outbank_ref[...] = bank_ref[...].T


def _make_sc_copy(size, dim, batch):
    info = plsc.get_sparse_core_info()
    nw = info.num_cores * info.num_subcores
    rows = size // nw
    crows = 256
    nchunks = rows // crows
    mesh = plsc.VectorSubcoreMesh(core_axis_name="c", subcore_axis_name="s")

    @functools.partial(
        pl.kernel,
        mesh=mesh,
        out_type=jax.ShapeDtypeStruct((size, dim), jnp.float32),
        scratch_types=[
            pltpu.VMEM((16,), jnp.int32),
            pltpu.VMEM((crows, dim), jnp.float32),
            pltpu.VMEM((crows, dim), jnp.float32),
            pltpu.SemaphoreType.DMA,
            pltpu.SemaphoreType.DMA,
            pltpu.SemaphoreType.DMA,
            pltpu.SemaphoreType.DMA,
        ],
    )
    def sc_copy(u_hbm, out_hbm, bank_hbm, new_hbm, u_v,
                buf0, buf1, si0, si1, so0, so1):
        wid = lax.axis_index("s") * info.num_cores + lax.axis_index("c")
        base = wid * rows
        pltpu.sync_copy(u_hbm, u_v)
        upd = u_v[...][0]
        bufs = (buf0, buf1)
        sin = (si0, si1)
        sout = (so0, so1)

        def staged_copy(src):
            in_cp = [None, None]
            out_cp = [None, None]
            for c in range(nchunks + 1):
                b = c % 2
                if c < nchunks:
                    if c >= 2:
                        out_cp[b].wait()
                    in_cp[b] = pltpu.async_copy(
                        src.at[pl.ds(base + c * crows, crows)], bufs[b], sin[b])
                if c >= 1:
                    pb = (c - 1) % 2
                    in_cp[pb].wait()
                    out_cp[pb] = pltpu.async_copy(
                        bufs[pb], new_hbm.at[pl.ds(base + (c - 1) * crows, crows)],
                        sout[pb])
            out_cp[(nchunks - 1) % 2].wait()
            if nchunks >= 2:
                out_cp[nchunks % 2].wait()

        lax.cond((upd != 0) & (base < batch),
                 lambda: staged_copy(out_hbm),
                 lambda: staged_copy(bank_hbm))

    return sc_copy


def kernel(output, bank, update):
    size, dim = bank.shape
    batch = output.shape[0]
    u16 = jnp.full((16,), jnp.asarray(update, jnp.int32))
    new_bank = _make_sc_copy(size, dim, batch)(u16, output, bank)
    r = 2048
    out_bank = pl.pallas_call(
        _tr_kernel,
        grid=(size // r,),
        in_specs=[pl.BlockSpec((r, dim), lambda i: (i, 0))],
        out_specs=pl.BlockSpec((dim, r), lambda i: (0, i)),
        out_shape=jax.ShapeDtypeStruct((dim, size), bank.dtype),
    )(bank)
    return (output, out_bank, new_bank)


# speedup vs baseline: 27.5825x; 1.8471x over previous
import functools

import jax
import jax.numpy as jnp
from jax.experimental import pallas as pl
from jax.experimental.pallas import tpu as pltpu


def _mb_kernel(nb_out, u_ref, out_in_ref, bank_ref,
               out_copy_ref, outbank_ref, newbank_ref):
    i = pl.program_id(0)
    blk = bank_ref[...]
    outbank_ref[...] = blk.T
    upd = (u_ref[0] != 0) & (i < nb_out)
    newbank_ref[...] = jnp.where(upd, out_in_ref[...], blk)

    @pl.when(i < nb_out)
    def _():
        out_copy_ref[...] = out_in_ref[...]


def kernel(output, bank, update):
    size, dim = bank.shape
    batch = output.shape[0]
    r = 4096
    nb_out = batch // r
    grid = size // r
    u = jnp.asarray(update, jnp.int32).reshape(1)

    body = functools.partial(_mb_kernel, nb_out)
    out_copy, out_bank, new_bank = pl.pallas_call(
        body,
        grid=(grid,),
        in_specs=[
            pl.BlockSpec(memory_space=pltpu.SMEM),
            pl.BlockSpec((r, dim), lambda i: (jnp.minimum(i, nb_out - 1), 0)),
            pl.BlockSpec((r, dim), lambda i: (i, 0)),
        ],
        out_specs=[
            pl.BlockSpec((r, dim), lambda i: (jnp.minimum(i, nb_out - 1), 0)),
            pl.BlockSpec((dim, r), lambda i: (0, i)),
            pl.BlockSpec((r, dim), lambda i: (i, 0)),
        ],
        out_shape=[
            jax.ShapeDtypeStruct((batch, dim), output.dtype),
            jax.ShapeDtypeStruct((dim, size), bank.dtype),
            jax.ShapeDtypeStruct((size, dim), bank.dtype),
        ],
    )(u, output, bank)
    return (out_copy, out_bank, new_bank)


# speedup vs baseline: 28.9586x; 1.0499x over previous
import functools

import jax
import jax.numpy as jnp
from jax.experimental import pallas as pl
from jax.experimental.pallas import tpu as pltpu


def _mb_kernel(batch, r, u_ref, out_in_ref, bank_ref,
               out_copy_ref, outbank_ref, newbank_ref):
    i = pl.program_id(0)
    blk = bank_ref[...]
    outbank_ref[...] = blk.T
    upd = u_ref[0] != 0

    @pl.when(i == 0)
    def _():
        out_full = out_in_ref[...]
        out_copy_ref[...] = out_full
        head = jnp.where(upd, out_full, blk[:batch])
        if r > batch:
            newbank_ref[...] = jnp.concatenate([head, blk[batch:]], axis=0)
        else:
            newbank_ref[...] = head

    @pl.when(i != 0)
    def _():
        newbank_ref[...] = blk


def kernel(output, bank, update):
    size, dim = bank.shape
    batch = output.shape[0]
    r = 8192
    grid = size // r
    u = jnp.asarray(update, jnp.int32).reshape(1)

    body = functools.partial(_mb_kernel, batch, r)
    out_copy, out_bank, new_bank = pl.pallas_call(
        body,
        grid=(grid,),
        in_specs=[
            pl.BlockSpec(memory_space=pltpu.SMEM),
            pl.BlockSpec((batch, dim), lambda i: (0, 0)),
            pl.BlockSpec((r, dim), lambda i: (i, 0)),
        ],
        out_specs=[
            pl.BlockSpec((batch, dim), lambda i: (0, 0)),
            pl.BlockSpec((dim, r), lambda i: (0, i)),
            pl.BlockSpec((r, dim), lambda i: (i, 0)),
        ],
        out_shape=[
            jax.ShapeDtypeStruct((batch, dim), output.dtype),
            jax.ShapeDtypeStruct((dim, size), bank.dtype),
            jax.ShapeDtypeStruct((size, dim), bank.dtype),
        ],
    )(u, output, bank)
    return (out_copy, out_bank, new_bank)


# speedup vs baseline: 29.8937x; 1.0323x over previous
import functools

import jax
import jax.numpy as jnp
from jax.experimental import pallas as pl
from jax.experimental.pallas import tpu as pltpu


def _mb_kernel(batch, r, u_ref, out_in_ref, bank_ref,
               outbank_ref, newbank_ref):
    i = pl.program_id(0)
    blk = bank_ref[...]
    outbank_ref[...] = blk.T
    upd = u_ref[0] != 0

    @pl.when(i == 0)
    def _():
        head = jnp.where(upd, out_in_ref[...], blk[:batch])
        if r > batch:
            newbank_ref[...] = jnp.concatenate([head, blk[batch:]], axis=0)
        else:
            newbank_ref[...] = head

    @pl.when(i != 0)
    def _():
        newbank_ref[...] = blk


def kernel(output, bank, update):
    size, dim = bank.shape
    batch = output.shape[0]
    r = 16384
    grid = size // r
    u = jnp.asarray(update, jnp.int32).reshape(1)

    body = functools.partial(_mb_kernel, batch, r)
    out_bank, new_bank = pl.pallas_call(
        body,
        grid=(grid,),
        in_specs=[
            pl.BlockSpec(memory_space=pltpu.SMEM),
            pl.BlockSpec((batch, dim), lambda i: (0, 0)),
            pl.BlockSpec((r, dim), lambda i: (i, 0)),
        ],
        out_specs=[
            pl.BlockSpec((dim, r), lambda i: (0, i)),
            pl.BlockSpec((r, dim), lambda i: (i, 0)),
        ],
        out_shape=[
            jax.ShapeDtypeStruct((dim, size), bank.dtype),
            jax.ShapeDtypeStruct((size, dim), bank.dtype),
        ],
    )(u, output, bank)
    return (output, out_bank, new_bank)
